# Initial kernel scaffold; baseline (speedup 1.0000x reference)
import jax
import jax.numpy as jnp
from jax.experimental import pallas as pl


def kernel(xs_pad, conv_w, proj, codebook):
    raise NotImplementedError("write your pallas kernel here")


# speedup vs baseline: 1.0250x; 1.0250x over previous
import numpy as np
import jax
import jax.numpy as jnp
from jax.experimental import pallas as pl

_SR = 16000
_FRAME_LEN = 400
_SHIFT = 160
_NFFT = 512
_NMEL = 128
_FBANK_MEAN = 15.41663
_FBANK_STD = 6.55582
_PATCH = 16
_EMBED = 512
_QN = 1024
_QD = 256
_NF = 992

_HIGHEST = jax.lax.Precision.HIGHEST


def _mel_fb():
    def hz2mel(h):
        return 1127.0 * np.log(1.0 + h / 700.0)
    low, high = 20.0, _SR / 2.0
    mel_pts = np.linspace(hz2mel(low), hz2mel(high), _NMEL + 2)
    bins = _NFFT // 2 + 1
    fft_mel = hz2mel(np.arange(bins) * _SR / _NFFT)
    fb = np.zeros((_NMEL, bins), dtype=np.float32)
    for m in range(_NMEL):
        l, c, r = mel_pts[m], mel_pts[m + 1], mel_pts[m + 2]
        up = (fft_mel - l) / (c - l)
        down = (r - fft_mel) / (r - c)
        fb[m] = np.maximum(0.0, np.minimum(up, down)).astype(np.float32)
    return fb


_MEL_FB = jnp.asarray(_mel_fb())
_WIN = jnp.asarray(((0.5 - 0.5 * np.cos(2 * np.pi * np.arange(_FRAME_LEN)
                                        / (_FRAME_LEN - 1))) ** 0.85).astype(np.float32))


def _fbank(wav):
    n_frames = 1 + (wav.shape[0] - _FRAME_LEN) // _SHIFT
    idx = np.arange(_FRAME_LEN)[None, :] + _SHIFT * np.arange(n_frames)[:, None]
    frames = wav[idx]
    frames = frames - jnp.mean(frames, axis=1, keepdims=True)
    prev = jnp.concatenate([frames[:, :1], frames[:, :-1]], axis=1)
    frames = frames - 0.97 * prev
    frames = frames * _WIN[None, :]
    spec = jnp.fft.rfft(frames, n=_NFFT, axis=1)
    power = jnp.abs(spec) ** 2
    mel = power @ _MEL_FB.T
    return jnp.log(jnp.maximum(mel, 1.1920928955078125e-07))


def _body(fb_ref, wconv_ref, proj_ref, cb_ref, out_ref):
    logmel = fb_ref[0, :_NF, :]
    pm = logmel.reshape(62, 16, 8, 16).transpose(0, 2, 1, 3).reshape(496, 256)
    f = jnp.dot(pm.astype(jnp.bfloat16), wconv_ref[...].astype(jnp.bfloat16),
                preferred_element_type=jnp.float32)
    mu = jnp.mean(f, axis=1, keepdims=True)
    var = jnp.mean((f - mu) ** 2, axis=1, keepdims=True)
    f = (f - mu) / jnp.sqrt(var + 1e-5)
    v = jnp.dot(f.astype(jnp.bfloat16), proj_ref[...].astype(jnp.bfloat16),
                preferred_element_type=jnp.float32)
    vn = v / (jnp.sqrt(jnp.sum(v * v, axis=1, keepdims=True)) + 1e-12)
    cb = cb_ref[...]
    cn = cb / (jnp.sqrt(jnp.sum(cb * cb, axis=1, keepdims=True)) + 1e-12)
    g = jax.lax.dot_general(vn.astype(jnp.bfloat16), cn.astype(jnp.bfloat16),
                            (((1,), (1,)), ((), ())),
                            preferred_element_type=jnp.float32)
    dist = (jnp.sum(vn * vn, axis=1, keepdims=True) - 2.0 * g
            + jnp.sum(cn * cn, axis=1)[None, :])
    dmin = jnp.min(dist, axis=1, keepdims=True)
    iota = jax.lax.broadcasted_iota(jnp.int32, dist.shape, 1)
    idx = jnp.min(jnp.where(dist <= dmin, iota, jnp.int32(1 << 30)), axis=1)
    out_ref[0, 0, :] = idx


def kernel(xs_pad, conv_w, proj, codebook):
    b = xs_pad.shape[0]
    wav = xs_pad * (2.0 ** 15)
    fbank = jax.vmap(_fbank)(wav)
    fbank = (fbank - _FBANK_MEAN) / (2.0 * _FBANK_STD)
    wconv = conv_w.reshape(_EMBED, _PATCH * _PATCH).T
    out = pl.pallas_call(
        _body,
        grid=(b,),
        in_specs=[
            pl.BlockSpec((1, 998, _NMEL), lambda i: (i, 0, 0)),
            pl.BlockSpec((_PATCH * _PATCH, _EMBED), lambda i: (0, 0)),
            pl.BlockSpec((_EMBED, _QD), lambda i: (0, 0)),
            pl.BlockSpec((_QN, _QD), lambda i: (0, 0)),
        ],
        out_specs=pl.BlockSpec((1, 1, 496), lambda i: (i, 0, 0)),
        out_shape=jax.ShapeDtypeStruct((b, 1, 496), jnp.int32),
    )(fbank, wconv, proj, codebook)
    return out.reshape(b, 496)


# speedup vs baseline: 1.0250x; 1.0000x over previous
import numpy as np
import jax
import jax.numpy as jnp
from jax.experimental import pallas as pl

_SR = 16000
_FRAME_LEN = 400
_SHIFT = 160
_NFFT = 512
_NMEL = 128
_FBANK_MEAN = 15.41663
_FBANK_STD = 6.55582
_PATCH = 16
_EMBED = 512
_QN = 1024
_QD = 256
_NF = 992


def _mel_fb():
    def hz2mel(h):
        return 1127.0 * np.log(1.0 + h / 700.0)
    low, high = 20.0, _SR / 2.0
    mel_pts = np.linspace(hz2mel(low), hz2mel(high), _NMEL + 2)
    bins = _NFFT // 2 + 1
    fft_mel = hz2mel(np.arange(bins) * _SR / _NFFT)
    fb = np.zeros((_NMEL, bins), dtype=np.float32)
    for m in range(_NMEL):
        l, c, r = mel_pts[m], mel_pts[m + 1], mel_pts[m + 2]
        up = (fft_mel - l) / (c - l)
        down = (r - fft_mel) / (r - c)
        fb[m] = np.maximum(0.0, np.minimum(up, down)).astype(np.float32)
    return fb


_MEL_FB = jnp.asarray(_mel_fb())
_WIN = jnp.asarray(((0.5 - 0.5 * np.cos(2 * np.pi * np.arange(_FRAME_LEN)
                                        / (_FRAME_LEN - 1))) ** 0.85).astype(np.float32))


def _fbank(wav):
    n_frames = 1 + (wav.shape[0] - _FRAME_LEN) // _SHIFT
    idx = np.arange(_FRAME_LEN)[None, :] + _SHIFT * np.arange(n_frames)[:, None]
    frames = wav[idx]
    frames = frames - jnp.mean(frames, axis=1, keepdims=True)
    prev = jnp.concatenate([frames[:, :1], frames[:, :-1]], axis=1)
    frames = frames - 0.97 * prev
    frames = frames * _WIN[None, :]
    spec = jnp.fft.rfft(frames, n=_NFFT, axis=1)
    power = jnp.abs(spec) ** 2
    mel = power @ _MEL_FB.T
    return jnp.log(jnp.maximum(mel, 1.1920928955078125e-07))


def _body(fb_ref, wconv_ref, proj_ref, cb_ref, out_ref):
    logmel = fb_ref[0, :_NF, :]
    pm = logmel.reshape(62, 16, 8, 16).transpose(0, 2, 1, 3).reshape(496, 256)
    f = jnp.dot(pm.astype(jnp.bfloat16), wconv_ref[...].astype(jnp.bfloat16),
                preferred_element_type=jnp.float32)
    mu = jnp.mean(f, axis=1, keepdims=True)
    var = jnp.mean((f - mu) ** 2, axis=1, keepdims=True)
    f = (f - mu) / jnp.sqrt(var + 1e-5)
    v = jnp.dot(f.astype(jnp.bfloat16), proj_ref[...].astype(jnp.bfloat16),
                preferred_element_type=jnp.float32)
    vn = v / (jnp.sqrt(jnp.sum(v * v, axis=1, keepdims=True)) + 1e-12)
    cb = cb_ref[...]
    cn = cb / (jnp.sqrt(jnp.sum(cb * cb, axis=1, keepdims=True)) + 1e-12)
    g = jax.lax.dot_general(vn.astype(jnp.bfloat16), cn.astype(jnp.bfloat16),
                            (((1,), (1,)), ((), ())),
                            preferred_element_type=jnp.float32)
    dist = (jnp.sum(vn * vn, axis=1, keepdims=True) - 2.0 * g
            + jnp.sum(cn * cn, axis=1)[None, :])
    dmin = jnp.min(dist, axis=1, keepdims=True)
    iota = jax.lax.broadcasted_iota(jnp.int32, dist.shape, 1)
    idx = jnp.min(jnp.where(dist <= dmin, iota, jnp.int32(1 << 30)), axis=1)
    out_ref[0, 0, :] = idx


def kernel(xs_pad, conv_w, proj, codebook):
    b = xs_pad.shape[0]
    wav = xs_pad * (2.0 ** 15)
    fbank = jax.vmap(_fbank)(wav)
    fbank = (fbank - _FBANK_MEAN) / (2.0 * _FBANK_STD)
    wconv = conv_w.reshape(_EMBED, _PATCH * _PATCH).T
    out = pl.pallas_call(
        _body,
        grid=(b,),
        in_specs=[
            pl.BlockSpec((1, 998, _NMEL), lambda i: (i, 0, 0)),
            pl.BlockSpec((_PATCH * _PATCH, _EMBED), lambda i: (0, 0)),
            pl.BlockSpec((_EMBED, _QD), lambda i: (0, 0)),
            pl.BlockSpec((_QN, _QD), lambda i: (0, 0)),
        ],
        out_specs=pl.BlockSpec((1, 1, 496), lambda i: (i, 0, 0)),
        out_shape=jax.ShapeDtypeStruct((b, 1, 496), jnp.int32),
    )(fbank, wconv, proj, codebook)
    return out.reshape(b, 496)


# speedup vs baseline: 1.0253x; 1.0003x over previous
import numpy as np
import jax
import jax.numpy as jnp
from jax.experimental import pallas as pl

_SR = 16000
_FRAME_LEN = 400
_SHIFT = 160
_NFFT = 512
_NMEL = 128
_FBANK_MEAN = 15.41663
_FBANK_STD = 6.55582
_PATCH = 16
_EMBED = 512
_QN = 1024
_QD = 256
_NF = 992


def _mel_fb():
    def hz2mel(h):
        return 1127.0 * np.log(1.0 + h / 700.0)
    low, high = 20.0, _SR / 2.0
    mel_pts = np.linspace(hz2mel(low), hz2mel(high), _NMEL + 2)
    bins = _NFFT // 2 + 1
    fft_mel = hz2mel(np.arange(bins) * _SR / _NFFT)
    fb = np.zeros((_NMEL, bins), dtype=np.float32)
    for m in range(_NMEL):
        l, c, r = mel_pts[m], mel_pts[m + 1], mel_pts[m + 2]
        up = (fft_mel - l) / (c - l)
        down = (r - fft_mel) / (r - c)
        fb[m] = np.maximum(0.0, np.minimum(up, down)).astype(np.float32)
    return fb


_MEL_FB = jnp.asarray(_mel_fb())
_WIN = jnp.asarray(((0.5 - 0.5 * np.cos(2 * np.pi * np.arange(_FRAME_LEN)
                                        / (_FRAME_LEN - 1))) ** 0.85).astype(np.float32))


def _fbank(wav):
    n_frames = 1 + (wav.shape[0] - _FRAME_LEN) // _SHIFT
    idx = np.arange(_FRAME_LEN)[None, :] + _SHIFT * np.arange(n_frames)[:, None]
    frames = wav[idx]
    frames = frames - jnp.mean(frames, axis=1, keepdims=True)
    prev = jnp.concatenate([frames[:, :1], frames[:, :-1]], axis=1)
    frames = frames - 0.97 * prev
    frames = frames * _WIN[None, :]
    spec = jnp.fft.rfft(frames, n=_NFFT, axis=1)
    power = jnp.abs(spec) ** 2
    mel = power @ _MEL_FB.T
    return jnp.log(jnp.maximum(mel, 1.1920928955078125e-07))


def _body(fb_ref, wconv_ref, proj_ref, cb_ref, out_ref):
    logmel = fb_ref[:, :_NF, :]
    pm = (logmel.reshape(2, 62, 16, 8, 16).transpose(0, 1, 3, 2, 4)
          .reshape(992, 256))
    f = jnp.dot(pm.astype(jnp.bfloat16), wconv_ref[...].astype(jnp.bfloat16),
                preferred_element_type=jnp.float32)
    mu = jnp.mean(f, axis=1, keepdims=True)
    var = jnp.mean((f - mu) ** 2, axis=1, keepdims=True)
    f = (f - mu) / jnp.sqrt(var + 1e-5)
    v = jnp.dot(f.astype(jnp.bfloat16), proj_ref[...].astype(jnp.bfloat16),
                preferred_element_type=jnp.float32)
    vn = v / (jnp.sqrt(jnp.sum(v * v, axis=1, keepdims=True)) + 1e-12)
    cb = cb_ref[...]
    cn = cb / (jnp.sqrt(jnp.sum(cb * cb, axis=1, keepdims=True)) + 1e-12)
    g = jax.lax.dot_general(vn.astype(jnp.bfloat16), cn.astype(jnp.bfloat16),
                            (((1,), (1,)), ((), ())),
                            preferred_element_type=jnp.float32)
    dist = (jnp.sum(vn * vn, axis=1, keepdims=True) - 2.0 * g
            + jnp.sum(cn * cn, axis=1)[None, :])
    dmin = jnp.min(dist, axis=1, keepdims=True)
    iota = jax.lax.broadcasted_iota(jnp.int32, dist.shape, 1)
    idx = jnp.min(jnp.where(dist <= dmin, iota, jnp.int32(1 << 30)), axis=1)
    out_ref[...] = idx.reshape(2, 1, 496)


def kernel(xs_pad, conv_w, proj, codebook):
    b = xs_pad.shape[0]
    wav = xs_pad * (2.0 ** 15)
    fbank = jax.vmap(_fbank)(wav)
    fbank = (fbank - _FBANK_MEAN) / (2.0 * _FBANK_STD)
    wconv = conv_w.reshape(_EMBED, _PATCH * _PATCH).T
    out = pl.pallas_call(
        _body,
        grid=(b // 2,),
        in_specs=[
            pl.BlockSpec((2, 998, _NMEL), lambda i: (i, 0, 0)),
            pl.BlockSpec((_PATCH * _PATCH, _EMBED), lambda i: (0, 0)),
            pl.BlockSpec((_EMBED, _QD), lambda i: (0, 0)),
            pl.BlockSpec((_QN, _QD), lambda i: (0, 0)),
        ],
        out_specs=pl.BlockSpec((2, 1, 496), lambda i: (i, 0, 0)),
        out_shape=jax.ShapeDtypeStruct((b, 1, 496), jnp.int32),
    )(fbank, wconv, proj, codebook)
    return out.reshape(b, 496)
